# baseline (speedup 1.0000x reference)
import jax
import jax.numpy as jnp
from jax.experimental import pallas as pl
from jax.experimental.pallas import tpu as pltpu

D_MODEL = 1024
E = 64
B = 64


def _gating_kernel(x_ref, gw_ref, gb_ref, g_out_ref):
    x = x_ref[...]
    gw = gw_ref[...]
    logits = jax.lax.dot_general(
        x, gw, (((1,), (1,)), ((), ())), preferred_element_type=jnp.float32
    ) + gb_ref[...]
    z = logits - jnp.max(logits, axis=1, keepdims=True)
    ez = jnp.exp(z)
    g = ez / jnp.sum(ez, axis=1, keepdims=True)
    m1 = jnp.max(g, axis=1, keepdims=True)
    g_wo_top1 = jnp.where(g == m1, -1.0, g)
    m2 = jnp.max(g_wo_top1, axis=1, keepdims=True)
    sel = g >= m2
    active = jnp.max(sel.astype(jnp.float32), axis=0, keepdims=True)
    g_out_ref[...] = g * active


def _expert_mm_kernel(x_ref, g_ref, w_ref, o_ref):
    e = pl.program_id(1)
    part = jnp.dot(x_ref[...], w_ref[...], preferred_element_type=jnp.float32)
    onehot = (jax.lax.broadcasted_iota(jnp.int32, (E, 1), 0) == e).astype(jnp.float32)
    col = jnp.dot(g_ref[...], onehot, preferred_element_type=jnp.float32)
    contrib = part * col
    o_ref[...] = jnp.where(e == 0, contrib, o_ref[...] + contrib)


def kernel(x, weight, gate_w, gate_b):
    gb2 = gate_b.reshape(1, E)

    g_eff = pl.pallas_call(
        _gating_kernel,
        out_shape=jax.ShapeDtypeStruct((B, E), jnp.float32),
    )(x, gate_w, gb2)

    HALF = D_MODEL // 2
    out = pl.pallas_call(
        _expert_mm_kernel,
        grid=(2, E),
        in_specs=[
            pl.BlockSpec((B, D_MODEL), lambda j, e: (0, 0)),
            pl.BlockSpec((B, E), lambda j, e: (0, 0)),
            pl.BlockSpec((D_MODEL, HALF), lambda j, e: (0, 2 * e + j)),
        ],
        out_specs=pl.BlockSpec((B, HALF), lambda j, e: (0, j)),
        out_shape=jax.ShapeDtypeStruct((B, D_MODEL), jnp.float32),
        compiler_params=pltpu.CompilerParams(
            dimension_semantics=("parallel", "arbitrary"),
        ),
    )(x, g_eff, weight)
    return out


# speedup vs baseline: 1.5750x; 1.5750x over previous
import jax
import jax.numpy as jnp
from jax.experimental import pallas as pl
from jax.experimental.pallas import tpu as pltpu

D_MODEL = 1024
E = 64
B = 64


def _gating_kernel(x_ref, gw_ref, gb_ref, g_out_ref):
    x = x_ref[...]
    gw = gw_ref[...]
    logits = jax.lax.dot_general(
        x, gw, (((1,), (1,)), ((), ())), preferred_element_type=jnp.float32
    ) + gb_ref[...]
    z = logits - jnp.max(logits, axis=1, keepdims=True)
    ez = jnp.exp(z)
    g = ez / jnp.sum(ez, axis=1, keepdims=True)
    m1 = jnp.max(g, axis=1, keepdims=True)
    g_wo_top1 = jnp.where(g == m1, -1.0, g)
    m2 = jnp.max(g_wo_top1, axis=1, keepdims=True)
    sel = g >= m2
    active = jnp.max(sel.astype(jnp.float32), axis=0, keepdims=True)
    g_out_ref[...] = g * active


def _expert_mm_kernel(x_ref, g_ref, w_ref, o_ref):
    i = pl.program_id(0)
    part = jnp.dot(x_ref[...], w_ref[...], preferred_element_type=jnp.float32)
    iota = jax.lax.broadcasted_iota(jnp.int32, (E, 2), 0)
    onehot = (iota == 2 * i + jax.lax.broadcasted_iota(jnp.int32, (E, 2), 1)).astype(
        jnp.float32
    )
    cols = jnp.dot(g_ref[...], onehot, preferred_element_type=jnp.float32)
    contrib = part[:, :D_MODEL] * cols[:, 0:1] + part[:, D_MODEL:] * cols[:, 1:2]
    o_ref[...] = jnp.where(i == 0, contrib, o_ref[...] + contrib)


def kernel(x, weight, gate_w, gate_b):
    gb2 = gate_b.reshape(1, E)

    g_eff = pl.pallas_call(
        _gating_kernel,
        out_shape=jax.ShapeDtypeStruct((B, E), jnp.float32),
    )(x, gate_w, gb2)

    out = pl.pallas_call(
        _expert_mm_kernel,
        grid=(E // 2,),
        in_specs=[
            pl.BlockSpec((B, D_MODEL), lambda i: (0, 0)),
            pl.BlockSpec((B, E), lambda i: (0, 0)),
            pl.BlockSpec((D_MODEL, 2 * D_MODEL), lambda i: (0, i)),
        ],
        out_specs=pl.BlockSpec((B, D_MODEL), lambda i: (0, 0)),
        out_shape=jax.ShapeDtypeStruct((B, D_MODEL), jnp.float32),
        compiler_params=pltpu.CompilerParams(
            dimension_semantics=("arbitrary",),
        ),
    )(x, g_eff, weight)
    return out


# speedup vs baseline: 1.6245x; 1.0314x over previous
import jax
import jax.numpy as jnp
from jax.experimental import pallas as pl
from jax.experimental.pallas import tpu as pltpu

D_MODEL = 1024
E = 64
B = 64
GRP = 4


def _gating_kernel(x_ref, gw_ref, gb_ref, g_out_ref):
    x = x_ref[...]
    gw = gw_ref[...]
    logits = jax.lax.dot_general(
        x, gw, (((1,), (1,)), ((), ())), preferred_element_type=jnp.float32
    ) + gb_ref[...]
    z = logits - jnp.max(logits, axis=1, keepdims=True)
    ez = jnp.exp(z)
    g = ez / jnp.sum(ez, axis=1, keepdims=True)
    m1 = jnp.max(g, axis=1, keepdims=True)
    g_wo_top1 = jnp.where(g == m1, -1.0, g)
    m2 = jnp.max(g_wo_top1, axis=1, keepdims=True)
    sel = g >= m2
    active = jnp.max(sel.astype(jnp.float32), axis=0, keepdims=True)
    g_out_ref[...] = g * active


def _expert_mm_kernel(x_ref, g_ref, w_ref, o_ref):
    i = pl.program_id(0)
    part = jnp.dot(x_ref[...], w_ref[...], preferred_element_type=jnp.float32)
    iota = jax.lax.broadcasted_iota(jnp.int32, (E, GRP), 0)
    onehot = (iota == GRP * i + jax.lax.broadcasted_iota(jnp.int32, (E, GRP), 1)).astype(
        jnp.float32
    )
    cols = jnp.dot(g_ref[...], onehot, preferred_element_type=jnp.float32)
    contrib = part[:, :D_MODEL] * cols[:, 0:1]
    for k in range(1, GRP):
        contrib += part[:, k * D_MODEL:(k + 1) * D_MODEL] * cols[:, k:k + 1]
    o_ref[...] = jnp.where(i == 0, contrib, o_ref[...] + contrib)


def kernel(x, weight, gate_w, gate_b):
    gb2 = gate_b.reshape(1, E)

    g_eff = pl.pallas_call(
        _gating_kernel,
        out_shape=jax.ShapeDtypeStruct((B, E), jnp.float32),
    )(x, gate_w, gb2)

    out = pl.pallas_call(
        _expert_mm_kernel,
        grid=(E // GRP,),
        in_specs=[
            pl.BlockSpec((B, D_MODEL), lambda i: (0, 0)),
            pl.BlockSpec((B, E), lambda i: (0, 0)),
            pl.BlockSpec((D_MODEL, GRP * D_MODEL), lambda i: (0, i)),
        ],
        out_specs=pl.BlockSpec((B, D_MODEL), lambda i: (0, 0)),
        out_shape=jax.ShapeDtypeStruct((B, D_MODEL), jnp.float32),
        compiler_params=pltpu.CompilerParams(
            dimension_semantics=("arbitrary",),
        ),
    )(x, g_eff, weight)
    return out
